# baseline (speedup 1.0000x reference)
import functools

import jax
import jax.numpy as jnp
from jax import lax
from jax.experimental import pallas as pl
from jax.experimental.pallas import tpu as pltpu
from jax.experimental.pallas import tpu_sc as plsc

NR = 50000
DIN = 128
H = 64
HH = 32
OUT = 32
E = 400000
NT = 16
TPR = 3136
PAD = NT * TPR
EPT = E // NT
EC = 200
NCH = EPT // EC
NCB = 25
NBLK = NCH // NCB
DR = 28
NDR = TPR // DR
BB = 2000
NB = NR // BB

_mesh = plsc.VectorSubcoreMesh(core_axis_name="c", subcore_axis_name="s")


def _sc_body(tabr3_hbm, tabp3_hbm,
             src_r2_hbm, dst_p2_hbm, src_p2_hbm, dst_r2_hbm,
             ones_hbm, zrows_hbm, flag_hbm,
             cnt3_hbm, aggp3_hbm, aggr3_hbm,
             idxs_v, idxd_v, r0_v, r1_v, r2_v, zstage_v, flag_v, acc_sh,
             g0, g1, g2, s0, s1, s2, zsem):
    c = lax.axis_index("c")
    s = lax.axis_index("s")
    rows = [r0_v, r1_v, r2_v]
    gsem = [g0, g1, g2]
    ssem = [s0, s1, s2]
    pltpu.sync_copy(zrows_hbm, zstage_v)
    pltpu.sync_copy(flag_hbm, flag_v)
    do_counts = jnp.max(flag_v[...])

    def zero_own():
        cps = [pltpu.async_copy(
                   zstage_v, acc_sh.at[pl.ds(s * TPR + j * DR, DR)], zsem)
               for j in range(NDR)]
        for cp in cps:
            cp.wait()

    def drain_own(out2):
        cps = [pltpu.async_copy(
                   acc_sh.at[pl.ds(s * TPR + j * DR, DR)],
                   out2.at[pl.ds(s * TPR + j * DR, DR)], zsem)
               for j in range(NDR)]
        for cp in cps:
            cp.wait()

    @pl.when(do_counts > 0)
    def _():
        pltpu.sync_copy(ones_hbm, r0_v)
        zero_own()
        plsc.subcore_barrier()

        def count_rel(dst2_hbm):
            def cblock(blk, carry):
                base = s * NCH + blk * NCB
                pltpu.sync_copy(dst2_hbm.at[pl.ds(base, NCB)], idxd_v)
                cps = [pltpu.async_copy(
                           r0_v, acc_sh.at[idxd_v.at[j]], ssem[j % 3],
                           add=True)
                       for j in range(NCB)]
                for cp in cps:
                    cp.wait()
                return carry
            lax.fori_loop(0, NBLK, cblock, 0)

        @pl.when(c == 0)
        def _():
            count_rel(dst_p2_hbm)

        @pl.when(c == 1)
        def _():
            count_rel(dst_r2_hbm)

        plsc.subcore_barrier()
        drain_own(cnt3_hbm.at[c])

    def one_relation(src2_hbm, dst2_hbm, tab3_hbm, out3_hbm):
        zero_own()
        plsc.subcore_barrier()
        tab2 = tab3_hbm.at[c]

        def block(blk, carry):
            base = s * NCH + blk * NCB
            pltpu.sync_copy(src2_hbm.at[pl.ds(base, NCB)], idxs_v)
            pltpu.sync_copy(dst2_hbm.at[pl.ds(base, NCB)], idxd_v)
            gd = {}
            sd = {}
            for j in range(2):
                gd[j] = pltpu.async_copy(
                    tab2.at[idxs_v.at[j]], rows[j], gsem[j])
            for j in range(NCB):
                gd[j].wait()
                sd[j] = pltpu.async_copy(
                    rows[j % 3], acc_sh.at[idxd_v.at[j]], ssem[j % 3],
                    add=True)
                nj = j + 2
                if nj < NCB:
                    if j >= 1:
                        sd[j - 1].wait()
                    gd[nj] = pltpu.async_copy(
                        tab2.at[idxs_v.at[nj]], rows[nj % 3], gsem[nj % 3])
            for j in range(NCB - 2, NCB):
                sd[j].wait()
            return carry

        lax.fori_loop(0, NBLK, block, 0)
        plsc.subcore_barrier()
        drain_own(out3_hbm.at[c])

    one_relation(src_r2_hbm, dst_p2_hbm, tabr3_hbm, aggp3_hbm)
    one_relation(src_p2_hbm, dst_r2_hbm, tabp3_hbm, aggr3_hbm)


_sc_call = functools.partial(
    pl.kernel,
    out_type=[jax.ShapeDtypeStruct((2, PAD, HH), jnp.float32)] * 3,
    mesh=_mesh,
    scratch_types=[
        pltpu.VMEM((NCB, EC), jnp.int32),
        pltpu.VMEM((NCB, EC), jnp.int32),
        pltpu.VMEM((EC, HH), jnp.float32),
        pltpu.VMEM((EC, HH), jnp.float32),
        pltpu.VMEM((EC, HH), jnp.float32),
        pltpu.VMEM((DR, HH), jnp.float32),
        pltpu.VMEM((16,), jnp.int32),
        pltpu.VMEM_SHARED((PAD, HH), jnp.float32),
        pltpu.SemaphoreType.DMA,
        pltpu.SemaphoreType.DMA,
        pltpu.SemaphoreType.DMA,
        pltpu.SemaphoreType.DMA,
        pltpu.SemaphoreType.DMA,
        pltpu.SemaphoreType.DMA,
        pltpu.SemaphoreType.DMA,
    ],
    compiler_params=pltpu.CompilerParams(use_tc_tiling_on_sc=False,
                                        needs_layout_passes=False),
)(_sc_body)


def _proj_body(xr_ref, xp_ref, wr_ref, br_ref, wp_ref, bp_ref,
               dp_ref, sp_ref,
               hr_ref, hp_ref, dpo_ref, spo_ref):
    hr = jnp.dot(xr_ref[...], wr_ref[...],
                 preferred_element_type=jnp.float32) + br_ref[...]
    hp = jnp.dot(xp_ref[...], wp_ref[...],
                 preferred_element_type=jnp.float32) + bp_ref[...]
    hr_ref[0] = hr[:, :HH]
    hr_ref[1] = hr[:, HH:]
    hp_ref[0] = hp[:, :HH]
    hp_ref[1] = hp[:, HH:]
    dpo_ref[...] = dp_ref[...] - NR
    spo_ref[...] = sp_ref[...] - NR


EB = (E // EC) // NB


def _proj(x_review, x_product, W_review, b_review, W_product, b_product,
          dst_p2_raw, src_p2_raw):
    blk = lambda: pl.BlockSpec((BB, DIN), lambda i: (i, 0))
    full = lambda shape: pl.BlockSpec(shape, lambda i: tuple(0 for _ in shape))
    outb = lambda: pl.BlockSpec((2, BB, HH), lambda i: (0, i, 0))
    eblk = lambda: pl.BlockSpec((EB, EC), lambda i: (i, 0))
    return pl.pallas_call(
        _proj_body,
        grid=(NB,),
        in_specs=[blk(), blk(), full((DIN, H)), full((1, H)),
                  full((DIN, H)), full((1, H)), eblk(), eblk()],
        out_specs=[outb(), outb(), eblk(), eblk()],
        out_shape=[jax.ShapeDtypeStruct((2, NR, HH), jnp.float32)] * 2
                  + [jax.ShapeDtypeStruct((E // EC, EC), jnp.int32)] * 2,
    )(x_review, x_product, W_review, b_review.reshape(1, H),
      W_product, b_product.reshape(1, H), dst_p2_raw, src_p2_raw)


def _combine_body(split_out, relu,
                  hr_ref, hp_ref, ap_ref, ar_ref, cnt_ref,
                  wroot_ref, wrel_ref, b_ref,
                  *out_refs):
    inv_p = 1.0 / jnp.maximum(cnt_ref[0][:, 0:1], 1.0)
    inv_r = 1.0 / jnp.maximum(cnt_ref[1][:, 0:1], 1.0)
    h_rev = jnp.concatenate([hr_ref[0], hr_ref[1]], axis=1)
    h_prod = jnp.concatenate([hp_ref[0], hp_ref[1]], axis=1)
    agg_p = jnp.concatenate([ap_ref[0], ap_ref[1]], axis=1) * inv_p
    agg_r = jnp.concatenate([ar_ref[0], ar_ref[1]], axis=1) * inv_r
    wroot = wroot_ref[...]
    b = b_ref[...]
    out_rev = (jnp.dot(h_rev, wroot, preferred_element_type=jnp.float32) + b
               + jnp.dot(agg_r, wrel_ref[1], preferred_element_type=jnp.float32))
    out_prod = (jnp.dot(h_prod, wroot, preferred_element_type=jnp.float32) + b
                + jnp.dot(agg_p, wrel_ref[0], preferred_element_type=jnp.float32))
    if relu:
        out_rev = jnp.maximum(out_rev, 0.0)
        out_prod = jnp.maximum(out_prod, 0.0)
    if split_out:
        out_refs[0][0] = out_rev[:, :HH]
        out_refs[0][1] = out_rev[:, HH:]
        out_refs[1][0] = out_prod[:, :HH]
        out_refs[1][1] = out_prod[:, HH:]
    else:
        out_refs[0][0] = out_rev
        out_refs[0][1] = out_prod


def _combine(hr3, hp3, aggp3, aggr3, cnt3, w_rel, w_root, b,
             split_out, relu):
    dout = w_root.shape[1]
    pblk = lambda w: pl.BlockSpec((2, BB, w), lambda i: (0, i, 0))
    full = lambda shape: pl.BlockSpec(shape, lambda i: tuple(0 for _ in shape))
    if split_out:
        out_specs = [pblk(HH), pblk(HH)]
        out_shape = [jax.ShapeDtypeStruct((2, NR, HH), jnp.float32)] * 2
    else:
        out_specs = [pl.BlockSpec((2, BB, dout), lambda i: (0, i, 0))]
        out_shape = [jax.ShapeDtypeStruct((2, NR, dout), jnp.float32)]
    return pl.pallas_call(
        functools.partial(_combine_body, split_out, relu),
        grid=(NB,),
        in_specs=[pblk(HH)] * 5
                 + [full((H, dout)), full((2, H, dout)), full((1, dout))],
        out_specs=out_specs,
        out_shape=out_shape,
    )(hr3, hp3, aggp3, aggr3, cnt3, w_root, w_rel, b.reshape(1, dout))


def kernel(x_review, x_product, edge_index_r2p, edge_index_p2r,
           W_review, b_review, W_product, b_product,
           conv1_w_rel, conv1_w_root, conv1_b,
           conv2_w_rel, conv2_w_root, conv2_b):
    src_r2 = edge_index_r2p[0].reshape(E // EC, EC)
    dst_p2_raw = edge_index_r2p[1].reshape(E // EC, EC)
    src_p2_raw = edge_index_p2r[0].reshape(E // EC, EC)
    dst_r2 = edge_index_p2r[1].reshape(E // EC, EC)
    ones = jnp.ones((EC, HH), jnp.float32)
    zrows = jnp.zeros((DR, HH), jnp.float32)
    flag1 = jnp.ones((16,), jnp.int32)
    flag0 = jnp.zeros((16,), jnp.int32)

    hr3, hp3, dst_p2, src_p2 = _proj(x_review, x_product, W_review, b_review,
                                     W_product, b_product,
                                     dst_p2_raw, src_p2_raw)
    cnt3, aggp3, aggr3 = _sc_call(hr3, hp3, src_r2, dst_p2, src_p2, dst_r2,
                                  ones, zrows, flag1)
    hr3b, hp3b = _combine(hr3, hp3, aggp3, aggr3, cnt3,
                          conv1_w_rel, conv1_w_root, conv1_b,
                          split_out=True, relu=True)
    _, aggp3b, aggr3b = _sc_call(hr3b, hp3b, src_r2, dst_p2, src_p2, dst_r2,
                                 ones, zrows, flag0)
    (out,) = _combine(hr3b, hp3b, aggp3b, aggr3b, cnt3,
                      conv2_w_rel, conv2_w_root, conv2_b,
                      split_out=False, relu=False)
    return out.reshape(2 * NR, OUT)


# speedup vs baseline: 1.0003x; 1.0003x over previous
import functools

import jax
import jax.numpy as jnp
from jax import lax
from jax.experimental import pallas as pl
from jax.experimental.pallas import tpu as pltpu
from jax.experimental.pallas import tpu_sc as plsc

NR = 50000
DIN = 128
H = 64
HH = 32
OUT = 32
E = 400000
NT = 16
TPR = 3136
PAD = NT * TPR
EPT = E // NT
EC = 200
NCH = EPT // EC
NCB = 25
NBLK = NCH // NCB
DR = 28
NDR = TPR // DR
BB = 2000
NB = NR // BB

_mesh = plsc.VectorSubcoreMesh(core_axis_name="c", subcore_axis_name="s")


def _sc_body(tabr3_hbm, tabp3_hbm,
             src_r2_hbm, dst_p2_hbm, src_p2_hbm, dst_r2_hbm,
             ones_hbm, zrows_hbm, flag_hbm,
             cnt3_hbm, aggp3_hbm, aggr3_hbm,
             idxs_v, idxd_v, r0_v, r1_v, r2_v, zstage_v, flag_v, acc_sh,
             g0, g1, g2, s0, s1, s2, zsem):
    c = lax.axis_index("c")
    s = lax.axis_index("s")
    rows = [r0_v, r1_v, r2_v]
    gsem = [g0, g1, g2]
    ssem = [s0, s1, s2]
    pltpu.sync_copy(zrows_hbm, zstage_v)
    pltpu.sync_copy(flag_hbm, flag_v)
    do_counts = jnp.max(flag_v[...])

    def zero_own():
        cps = [pltpu.async_copy(
                   zstage_v, acc_sh.at[pl.ds(s * TPR + j * DR, DR)], zsem)
               for j in range(NDR)]
        for cp in cps:
            cp.wait()

    def drain_own(out2):
        cps = [pltpu.async_copy(
                   acc_sh.at[pl.ds(s * TPR + j * DR, DR)],
                   out2.at[pl.ds(s * TPR + j * DR, DR)], zsem)
               for j in range(NDR)]
        for cp in cps:
            cp.wait()

    @pl.when(do_counts > 0)
    def _():
        pltpu.sync_copy(ones_hbm, r0_v)
        zero_own()
        plsc.subcore_barrier()

        def count_rel(dst2_hbm):
            def cblock(blk, carry):
                base = s * NCH + blk * NCB
                pltpu.sync_copy(dst2_hbm.at[pl.ds(base, NCB)], idxd_v)
                cps = [pltpu.async_copy(
                           r0_v, acc_sh.at[idxd_v.at[j]], ssem[j % 3],
                           add=True)
                       for j in range(NCB)]
                for cp in cps:
                    cp.wait()
                return carry
            lax.fori_loop(0, NBLK, cblock, 0)

        @pl.when(c == 0)
        def _():
            count_rel(dst_p2_hbm)

        @pl.when(c == 1)
        def _():
            count_rel(dst_r2_hbm)

        plsc.subcore_barrier()
        drain_own(cnt3_hbm.at[c])

    def one_relation(src2_hbm, dst2_hbm, tab3_hbm, out3_hbm):
        zero_own()
        plsc.subcore_barrier()
        tab2 = tab3_hbm.at[c]

        def block(blk, carry):
            base = s * NCH + blk * NCB
            pltpu.sync_copy(src2_hbm.at[pl.ds(base, NCB)], idxs_v)
            pltpu.sync_copy(dst2_hbm.at[pl.ds(base, NCB)], idxd_v)
            gd = {}
            sd = {}
            for j in range(2):
                gd[j] = pltpu.async_copy(
                    tab2.at[idxs_v.at[j]], rows[j], gsem[j])
            for j in range(NCB):
                gd[j].wait()
                sd[j] = pltpu.async_copy(
                    rows[j % 3], acc_sh.at[idxd_v.at[j]], ssem[j % 3],
                    add=True)
                nj = j + 2
                if nj < NCB:
                    if j >= 1:
                        sd[j - 1].wait()
                    gd[nj] = pltpu.async_copy(
                        tab2.at[idxs_v.at[nj]], rows[nj % 3], gsem[nj % 3])
            for j in range(NCB - 3, NCB):
                sd[j].wait()
            return carry

        lax.fori_loop(0, NBLK, block, 0)
        plsc.subcore_barrier()
        drain_own(out3_hbm.at[c])

    one_relation(src_r2_hbm, dst_p2_hbm, tabr3_hbm, aggp3_hbm)
    one_relation(src_p2_hbm, dst_r2_hbm, tabp3_hbm, aggr3_hbm)


_sc_call = functools.partial(
    pl.kernel,
    out_type=[jax.ShapeDtypeStruct((2, PAD, HH), jnp.float32)] * 3,
    mesh=_mesh,
    scratch_types=[
        pltpu.VMEM((NCB, EC), jnp.int32),
        pltpu.VMEM((NCB, EC), jnp.int32),
        pltpu.VMEM((EC, HH), jnp.float32),
        pltpu.VMEM((EC, HH), jnp.float32),
        pltpu.VMEM((EC, HH), jnp.float32),
        pltpu.VMEM((DR, HH), jnp.float32),
        pltpu.VMEM((16,), jnp.int32),
        pltpu.VMEM_SHARED((PAD, HH), jnp.float32),
        pltpu.SemaphoreType.DMA,
        pltpu.SemaphoreType.DMA,
        pltpu.SemaphoreType.DMA,
        pltpu.SemaphoreType.DMA,
        pltpu.SemaphoreType.DMA,
        pltpu.SemaphoreType.DMA,
        pltpu.SemaphoreType.DMA,
    ],
    compiler_params=pltpu.CompilerParams(use_tc_tiling_on_sc=False,
                                        needs_layout_passes=False),
)(_sc_body)


def _proj_body(xr_ref, xp_ref, wr_ref, br_ref, wp_ref, bp_ref,
               dp_ref, sp_ref,
               hr_ref, hp_ref, dpo_ref, spo_ref):
    hr = jnp.dot(xr_ref[...], wr_ref[...],
                 preferred_element_type=jnp.float32) + br_ref[...]
    hp = jnp.dot(xp_ref[...], wp_ref[...],
                 preferred_element_type=jnp.float32) + bp_ref[...]
    hr_ref[0] = hr[:, :HH]
    hr_ref[1] = hr[:, HH:]
    hp_ref[0] = hp[:, :HH]
    hp_ref[1] = hp[:, HH:]
    dpo_ref[...] = dp_ref[...] - NR
    spo_ref[...] = sp_ref[...] - NR


EB = (E // EC) // NB


def _proj(x_review, x_product, W_review, b_review, W_product, b_product,
          dst_p2_raw, src_p2_raw):
    blk = lambda: pl.BlockSpec((BB, DIN), lambda i: (i, 0))
    full = lambda shape: pl.BlockSpec(shape, lambda i: tuple(0 for _ in shape))
    outb = lambda: pl.BlockSpec((2, BB, HH), lambda i: (0, i, 0))
    eblk = lambda: pl.BlockSpec((EB, EC), lambda i: (i, 0))
    return pl.pallas_call(
        _proj_body,
        grid=(NB,),
        in_specs=[blk(), blk(), full((DIN, H)), full((1, H)),
                  full((DIN, H)), full((1, H)), eblk(), eblk()],
        out_specs=[outb(), outb(), eblk(), eblk()],
        out_shape=[jax.ShapeDtypeStruct((2, NR, HH), jnp.float32)] * 2
                  + [jax.ShapeDtypeStruct((E // EC, EC), jnp.int32)] * 2,
    )(x_review, x_product, W_review, b_review.reshape(1, H),
      W_product, b_product.reshape(1, H), dst_p2_raw, src_p2_raw)


def _combine_body(split_out, relu,
                  hr_ref, hp_ref, ap_ref, ar_ref, cnt_ref,
                  wroot_ref, wrel_ref, b_ref,
                  *out_refs):
    inv_p = 1.0 / jnp.maximum(cnt_ref[0][:, 0:1], 1.0)
    inv_r = 1.0 / jnp.maximum(cnt_ref[1][:, 0:1], 1.0)
    h_rev = jnp.concatenate([hr_ref[0], hr_ref[1]], axis=1)
    h_prod = jnp.concatenate([hp_ref[0], hp_ref[1]], axis=1)
    agg_p = jnp.concatenate([ap_ref[0], ap_ref[1]], axis=1) * inv_p
    agg_r = jnp.concatenate([ar_ref[0], ar_ref[1]], axis=1) * inv_r
    wroot = wroot_ref[...]
    b = b_ref[...]
    out_rev = (jnp.dot(h_rev, wroot, preferred_element_type=jnp.float32) + b
               + jnp.dot(agg_r, wrel_ref[1], preferred_element_type=jnp.float32))
    out_prod = (jnp.dot(h_prod, wroot, preferred_element_type=jnp.float32) + b
                + jnp.dot(agg_p, wrel_ref[0], preferred_element_type=jnp.float32))
    if relu:
        out_rev = jnp.maximum(out_rev, 0.0)
        out_prod = jnp.maximum(out_prod, 0.0)
    if split_out:
        out_refs[0][0] = out_rev[:, :HH]
        out_refs[0][1] = out_rev[:, HH:]
        out_refs[1][0] = out_prod[:, :HH]
        out_refs[1][1] = out_prod[:, HH:]
    else:
        out_refs[0][0] = out_rev
        out_refs[0][1] = out_prod


def _combine(hr3, hp3, aggp3, aggr3, cnt3, w_rel, w_root, b,
             split_out, relu):
    dout = w_root.shape[1]
    pblk = lambda w: pl.BlockSpec((2, BB, w), lambda i: (0, i, 0))
    full = lambda shape: pl.BlockSpec(shape, lambda i: tuple(0 for _ in shape))
    if split_out:
        out_specs = [pblk(HH), pblk(HH)]
        out_shape = [jax.ShapeDtypeStruct((2, NR, HH), jnp.float32)] * 2
    else:
        out_specs = [pl.BlockSpec((2, BB, dout), lambda i: (0, i, 0))]
        out_shape = [jax.ShapeDtypeStruct((2, NR, dout), jnp.float32)]
    return pl.pallas_call(
        functools.partial(_combine_body, split_out, relu),
        grid=(NB,),
        in_specs=[pblk(HH)] * 5
                 + [full((H, dout)), full((2, H, dout)), full((1, dout))],
        out_specs=out_specs,
        out_shape=out_shape,
    )(hr3, hp3, aggp3, aggr3, cnt3, w_root, w_rel, b.reshape(1, dout))


def kernel(x_review, x_product, edge_index_r2p, edge_index_p2r,
           W_review, b_review, W_product, b_product,
           conv1_w_rel, conv1_w_root, conv1_b,
           conv2_w_rel, conv2_w_root, conv2_b):
    src_r2 = edge_index_r2p[0].reshape(E // EC, EC)
    dst_p2_raw = edge_index_r2p[1].reshape(E // EC, EC)
    src_p2_raw = edge_index_p2r[0].reshape(E // EC, EC)
    dst_r2 = edge_index_p2r[1].reshape(E // EC, EC)
    ones = jnp.ones((EC, HH), jnp.float32)
    zrows = jnp.zeros((DR, HH), jnp.float32)
    flag1 = jnp.ones((16,), jnp.int32)
    flag0 = jnp.zeros((16,), jnp.int32)

    hr3, hp3, dst_p2, src_p2 = _proj(x_review, x_product, W_review, b_review,
                                     W_product, b_product,
                                     dst_p2_raw, src_p2_raw)
    cnt3, aggp3, aggr3 = _sc_call(hr3, hp3, src_r2, dst_p2, src_p2, dst_r2,
                                  ones, zrows, flag1)
    hr3b, hp3b = _combine(hr3, hp3, aggp3, aggr3, cnt3,
                          conv1_w_rel, conv1_w_root, conv1_b,
                          split_out=True, relu=True)
    _, aggp3b, aggr3b = _sc_call(hr3b, hp3b, src_r2, dst_p2, src_p2, dst_r2,
                                 ones, zrows, flag0)
    (out,) = _combine(hr3b, hp3b, aggp3b, aggr3b, cnt3,
                      conv2_w_rel, conv2_w_root, conv2_b,
                      split_out=False, relu=False)
    return out.reshape(2 * NR, OUT)


# speedup vs baseline: 1.0169x; 1.0166x over previous
import functools

import jax
import jax.numpy as jnp
from jax import lax
from jax.experimental import pallas as pl
from jax.experimental.pallas import tpu as pltpu
from jax.experimental.pallas import tpu_sc as plsc

NR = 50000
DIN = 128
H = 64
HH = 32
OUT = 32
E = 400000
NT = 16
TPR = 3136
PAD = NT * TPR
EPT = E // NT
EC = 200
NCH = EPT // EC
NCB = 25
NBLK = NCH // NCB
DR = 28
NDR = TPR // DR
BB = 2000
NB = NR // BB

_mesh = plsc.VectorSubcoreMesh(core_axis_name="c", subcore_axis_name="s")


def _sc_body(tabr3_hbm, tabp3_hbm,
             src_r2_hbm, dst_p2_hbm, src_p2_hbm, dst_r2_hbm,
             ones_hbm, zrows_hbm, flag_hbm,
             cnt3_hbm, aggp3_hbm, aggr3_hbm,
             idxs_v, idxd_v, r0_v, r1_v, r2_v, zstage_v, flag_v, acc_sh,
             g0, g1, g2, s0, s1, s2, zsem):
    c = lax.axis_index("c")
    s = lax.axis_index("s")
    rows = [r0_v, r1_v, r2_v]
    gsem = [g0, g1, g2]
    ssem = [s0, s1, s2]
    pltpu.sync_copy(zrows_hbm, zstage_v)
    pltpu.sync_copy(flag_hbm, flag_v)
    do_counts = jnp.max(flag_v[...])

    def zero_own():
        cps = [pltpu.async_copy(
                   zstage_v, acc_sh.at[pl.ds(s * TPR + j * DR, DR)], zsem)
               for j in range(NDR)]
        for cp in cps:
            cp.wait()

    def drain_own(out2):
        cps = [pltpu.async_copy(
                   acc_sh.at[pl.ds(s * TPR + j * DR, DR)],
                   out2.at[pl.ds(s * TPR + j * DR, DR)], zsem)
               for j in range(NDR)]
        for cp in cps:
            cp.wait()

    @pl.when(do_counts > 0)
    def _():
        pltpu.sync_copy(ones_hbm, r0_v)
        zero_own()
        plsc.subcore_barrier()

        def count_rel(dst2_hbm):
            def cblock(blk, carry):
                base = s * NCH + blk * NCB
                pltpu.sync_copy(dst2_hbm.at[pl.ds(base, NCB)], idxd_v)
                cps = [pltpu.async_copy(
                           r0_v, acc_sh.at[idxd_v.at[j]], ssem[j % 3],
                           add=True)
                       for j in range(NCB)]
                for cp in cps:
                    cp.wait()
                return carry
            lax.fori_loop(0, NBLK, cblock, 0)

        @pl.when(c == 0)
        def _():
            count_rel(dst_p2_hbm)

        @pl.when(c == 1)
        def _():
            count_rel(dst_r2_hbm)

        plsc.subcore_barrier()
        drain_own(cnt3_hbm.at[c])

    def one_relation(src2_hbm, dst2_hbm, tab3_hbm, out3_hbm):
        zero_own()
        plsc.subcore_barrier()
        tab2 = tab3_hbm.at[c]

        def block(blk, carry):
            base = s * NCH + blk * NCB
            pltpu.sync_copy(src2_hbm.at[pl.ds(base, NCB)], idxs_v)
            pltpu.sync_copy(dst2_hbm.at[pl.ds(base, NCB)], idxd_v)
            gd = {}
            sd = {}
            for j in range(3):
                gd[j] = pltpu.async_copy(
                    tab2.at[idxs_v.at[j]], rows[j], gsem[j])
            for j in range(NCB):
                gd[j].wait()
                sd[j] = pltpu.async_copy(
                    rows[j % 3], acc_sh.at[idxd_v.at[j]], ssem[j % 3],
                    add=True)
                nj = j + 3
                if nj < NCB:
                    sd[j].wait()
                    gd[nj] = pltpu.async_copy(
                        tab2.at[idxs_v.at[nj]], rows[nj % 3], gsem[nj % 3])
            for j in range(NCB - 3, NCB):
                sd[j].wait()
            return carry

        lax.fori_loop(0, NBLK, block, 0)
        plsc.subcore_barrier()
        drain_own(out3_hbm.at[c])

    one_relation(src_r2_hbm, dst_p2_hbm, tabr3_hbm, aggp3_hbm)
    one_relation(src_p2_hbm, dst_r2_hbm, tabp3_hbm, aggr3_hbm)


_sc_call = functools.partial(
    pl.kernel,
    out_type=[jax.ShapeDtypeStruct((2, PAD, HH), jnp.float32)] * 3,
    mesh=_mesh,
    scratch_types=[
        pltpu.VMEM((NCB, EC), jnp.int32),
        pltpu.VMEM((NCB, EC), jnp.int32),
        pltpu.VMEM((EC, HH), jnp.float32),
        pltpu.VMEM((EC, HH), jnp.float32),
        pltpu.VMEM((EC, HH), jnp.float32),
        pltpu.VMEM((DR, HH), jnp.float32),
        pltpu.VMEM((16,), jnp.int32),
        pltpu.VMEM_SHARED((PAD, HH), jnp.float32),
        pltpu.SemaphoreType.DMA,
        pltpu.SemaphoreType.DMA,
        pltpu.SemaphoreType.DMA,
        pltpu.SemaphoreType.DMA,
        pltpu.SemaphoreType.DMA,
        pltpu.SemaphoreType.DMA,
        pltpu.SemaphoreType.DMA,
    ],
    compiler_params=pltpu.CompilerParams(use_tc_tiling_on_sc=False,
                                        needs_layout_passes=False),
)(_sc_body)


def _proj_body(xr_ref, xp_ref, wr_ref, br_ref, wp_ref, bp_ref,
               dp_ref, sp_ref,
               hr_ref, hp_ref, dpo_ref, spo_ref):
    hr = jnp.dot(xr_ref[...], wr_ref[...],
                 preferred_element_type=jnp.float32) + br_ref[...]
    hp = jnp.dot(xp_ref[...], wp_ref[...],
                 preferred_element_type=jnp.float32) + bp_ref[...]
    hr_ref[0] = hr[:, :HH]
    hr_ref[1] = hr[:, HH:]
    hp_ref[0] = hp[:, :HH]
    hp_ref[1] = hp[:, HH:]
    dpo_ref[...] = dp_ref[...] - NR
    spo_ref[...] = sp_ref[...] - NR


EB = (E // EC) // NB


def _proj(x_review, x_product, W_review, b_review, W_product, b_product,
          dst_p2_raw, src_p2_raw):
    blk = lambda: pl.BlockSpec((BB, DIN), lambda i: (i, 0))
    full = lambda shape: pl.BlockSpec(shape, lambda i: tuple(0 for _ in shape))
    outb = lambda: pl.BlockSpec((2, BB, HH), lambda i: (0, i, 0))
    eblk = lambda: pl.BlockSpec((EB, EC), lambda i: (i, 0))
    return pl.pallas_call(
        _proj_body,
        grid=(NB,),
        in_specs=[blk(), blk(), full((DIN, H)), full((1, H)),
                  full((DIN, H)), full((1, H)), eblk(), eblk()],
        out_specs=[outb(), outb(), eblk(), eblk()],
        out_shape=[jax.ShapeDtypeStruct((2, NR, HH), jnp.float32)] * 2
                  + [jax.ShapeDtypeStruct((E // EC, EC), jnp.int32)] * 2,
    )(x_review, x_product, W_review, b_review.reshape(1, H),
      W_product, b_product.reshape(1, H), dst_p2_raw, src_p2_raw)


def _combine_body(split_out, relu,
                  hr_ref, hp_ref, ap_ref, ar_ref, cnt_ref,
                  wroot_ref, wrel_ref, b_ref,
                  *out_refs):
    inv_p = 1.0 / jnp.maximum(cnt_ref[0][:, 0:1], 1.0)
    inv_r = 1.0 / jnp.maximum(cnt_ref[1][:, 0:1], 1.0)
    h_rev = jnp.concatenate([hr_ref[0], hr_ref[1]], axis=1)
    h_prod = jnp.concatenate([hp_ref[0], hp_ref[1]], axis=1)
    agg_p = jnp.concatenate([ap_ref[0], ap_ref[1]], axis=1) * inv_p
    agg_r = jnp.concatenate([ar_ref[0], ar_ref[1]], axis=1) * inv_r
    wroot = wroot_ref[...]
    b = b_ref[...]
    out_rev = (jnp.dot(h_rev, wroot, preferred_element_type=jnp.float32) + b
               + jnp.dot(agg_r, wrel_ref[1], preferred_element_type=jnp.float32))
    out_prod = (jnp.dot(h_prod, wroot, preferred_element_type=jnp.float32) + b
                + jnp.dot(agg_p, wrel_ref[0], preferred_element_type=jnp.float32))
    if relu:
        out_rev = jnp.maximum(out_rev, 0.0)
        out_prod = jnp.maximum(out_prod, 0.0)
    if split_out:
        out_refs[0][0] = out_rev[:, :HH]
        out_refs[0][1] = out_rev[:, HH:]
        out_refs[1][0] = out_prod[:, :HH]
        out_refs[1][1] = out_prod[:, HH:]
    else:
        out_refs[0][0] = out_rev
        out_refs[0][1] = out_prod


def _combine(hr3, hp3, aggp3, aggr3, cnt3, w_rel, w_root, b,
             split_out, relu):
    dout = w_root.shape[1]
    pblk = lambda w: pl.BlockSpec((2, BB, w), lambda i: (0, i, 0))
    full = lambda shape: pl.BlockSpec(shape, lambda i: tuple(0 for _ in shape))
    if split_out:
        out_specs = [pblk(HH), pblk(HH)]
        out_shape = [jax.ShapeDtypeStruct((2, NR, HH), jnp.float32)] * 2
    else:
        out_specs = [pl.BlockSpec((2, BB, dout), lambda i: (0, i, 0))]
        out_shape = [jax.ShapeDtypeStruct((2, NR, dout), jnp.float32)]
    return pl.pallas_call(
        functools.partial(_combine_body, split_out, relu),
        grid=(NB,),
        in_specs=[pblk(HH)] * 5
                 + [full((H, dout)), full((2, H, dout)), full((1, dout))],
        out_specs=out_specs,
        out_shape=out_shape,
    )(hr3, hp3, aggp3, aggr3, cnt3, w_root, w_rel, b.reshape(1, dout))


def kernel(x_review, x_product, edge_index_r2p, edge_index_p2r,
           W_review, b_review, W_product, b_product,
           conv1_w_rel, conv1_w_root, conv1_b,
           conv2_w_rel, conv2_w_root, conv2_b):
    src_r2 = edge_index_r2p[0].reshape(E // EC, EC)
    dst_p2_raw = edge_index_r2p[1].reshape(E // EC, EC)
    src_p2_raw = edge_index_p2r[0].reshape(E // EC, EC)
    dst_r2 = edge_index_p2r[1].reshape(E // EC, EC)
    ones = jnp.ones((EC, HH), jnp.float32)
    zrows = jnp.zeros((DR, HH), jnp.float32)
    flag1 = jnp.ones((16,), jnp.int32)
    flag0 = jnp.zeros((16,), jnp.int32)

    hr3, hp3, dst_p2, src_p2 = _proj(x_review, x_product, W_review, b_review,
                                     W_product, b_product,
                                     dst_p2_raw, src_p2_raw)
    cnt3, aggp3, aggr3 = _sc_call(hr3, hp3, src_r2, dst_p2, src_p2, dst_r2,
                                  ones, zrows, flag1)
    hr3b, hp3b = _combine(hr3, hp3, aggp3, aggr3, cnt3,
                          conv1_w_rel, conv1_w_root, conv1_b,
                          split_out=True, relu=True)
    _, aggp3b, aggr3b = _sc_call(hr3b, hp3b, src_r2, dst_p2, src_p2, dst_r2,
                                 ones, zrows, flag0)
    (out,) = _combine(hr3b, hp3b, aggp3b, aggr3b, cnt3,
                      conv2_w_rel, conv2_w_root, conv2_b,
                      split_out=False, relu=False)
    return out.reshape(2 * NR, OUT)


# speedup vs baseline: 1.0317x; 1.0145x over previous
import functools

import jax
import jax.numpy as jnp
from jax import lax
from jax.experimental import pallas as pl
from jax.experimental.pallas import tpu as pltpu
from jax.experimental.pallas import tpu_sc as plsc

NR = 50000
DIN = 128
H = 64
HH = 32
OUT = 32
E = 400000
NT = 16
TPR = 3136
PAD = NT * TPR
EPT = E // NT
EC = 200
NCH = EPT // EC
NCB = 25
NBLK = NCH // NCB
DR = 28
NDR = TPR // DR
BB = 2000
NB = NR // BB

_mesh = plsc.VectorSubcoreMesh(core_axis_name="c", subcore_axis_name="s")


def _sc_body(tabr3_hbm, tabp3_hbm,
             src_r2_hbm, dst_p2_hbm, src_p2_hbm, dst_r2_hbm,
             ones_hbm, zrows_hbm, flag_hbm,
             cnt3_hbm, aggp3_hbm, aggr3_hbm,
             idxs_v, idxd_v, r0_v, r1_v, r2_v, zstage_v, flag_v, acc_sh,
             g0, g1, g2, s0, s1, s2, zsem):
    c = lax.axis_index("c")
    s = lax.axis_index("s")
    rows = [r0_v, r1_v, r2_v]
    gsem = [g0, g1, g2]
    ssem = [s0, s1, s2]
    pltpu.sync_copy(zrows_hbm, zstage_v)
    pltpu.sync_copy(flag_hbm, flag_v)
    do_counts = jnp.max(flag_v[...])

    def zero_own():
        cps = [pltpu.async_copy(
                   zstage_v, acc_sh.at[pl.ds(s * TPR + j * DR, DR)], zsem)
               for j in range(NDR)]
        for cp in cps:
            cp.wait()

    def drain_own(out2):
        cps = [pltpu.async_copy(
                   acc_sh.at[pl.ds(s * TPR + j * DR, DR)],
                   out2.at[pl.ds(s * TPR + j * DR, DR)], zsem)
               for j in range(NDR)]
        for cp in cps:
            cp.wait()

    @pl.when(do_counts > 0)
    def _():
        pltpu.sync_copy(ones_hbm, r0_v)
        zero_own()
        plsc.subcore_barrier()

        def count_rel(dst2_hbm):
            def cblock(blk, carry):
                base = s * NCH + blk * NCB
                pltpu.sync_copy(dst2_hbm.at[pl.ds(base, NCB)], idxd_v)
                cps = [pltpu.async_copy(
                           r0_v, acc_sh.at[idxd_v.at[j]], ssem[j % 3],
                           add=True)
                       for j in range(NCB)]
                for cp in cps:
                    cp.wait()
                return carry
            lax.fori_loop(0, NBLK, cblock, 0)

        @pl.when(c == 0)
        def _():
            count_rel(dst_p2_hbm)

        @pl.when(c == 1)
        def _():
            count_rel(dst_r2_hbm)

        plsc.subcore_barrier()
        drain_own(cnt3_hbm.at[c])

    def one_relation(src2_hbm, dst2_hbm, tab3_hbm, out3_hbm):
        zero_own()
        plsc.subcore_barrier()
        tab2 = tab3_hbm.at[c]

        def block(blk, carry):
            base = s * NCH + blk * NCB
            pltpu.sync_copy(src2_hbm.at[pl.ds(base, NCB)], idxs_v)
            pltpu.sync_copy(dst2_hbm.at[pl.ds(base, NCB)], idxd_v)
            gd = {}
            sd = {}
            for j in range(3):
                gd[j] = pltpu.async_copy(
                    tab2.at[idxs_v.at[j]], rows[j], gsem[j])
            for j in range(NCB):
                gd[j].wait()
                sd[j] = pltpu.async_copy(
                    rows[j % 3], acc_sh.at[idxd_v.at[j]], ssem[j % 3],
                    add=True)
                nj = j + 3
                if nj < NCB:
                    sd[j].wait()
                    gd[nj] = pltpu.async_copy(
                        tab2.at[idxs_v.at[nj]], rows[nj % 3], gsem[nj % 3])
            for j in range(NCB - 3, NCB):
                sd[j].wait()
            return carry

        lax.fori_loop(0, NBLK, block, 0)
        plsc.subcore_barrier()
        drain_own(out3_hbm.at[c])

    one_relation(src_r2_hbm, dst_p2_hbm, tabr3_hbm, aggp3_hbm)
    one_relation(src_p2_hbm, dst_r2_hbm, tabp3_hbm, aggr3_hbm)


_sc_call = functools.partial(
    pl.kernel,
    out_type=[jax.ShapeDtypeStruct((2, PAD, HH), jnp.float32)] * 3,
    mesh=_mesh,
    scratch_types=[
        pltpu.VMEM((NCB, EC), jnp.int32),
        pltpu.VMEM((NCB, EC), jnp.int32),
        pltpu.VMEM((EC, HH), jnp.float32),
        pltpu.VMEM((EC, HH), jnp.float32),
        pltpu.VMEM((EC, HH), jnp.float32),
        pltpu.VMEM((DR, HH), jnp.float32),
        pltpu.VMEM((16,), jnp.int32),
        pltpu.VMEM_SHARED((PAD, HH), jnp.float32),
        pltpu.SemaphoreType.DMA,
        pltpu.SemaphoreType.DMA,
        pltpu.SemaphoreType.DMA,
        pltpu.SemaphoreType.DMA,
        pltpu.SemaphoreType.DMA,
        pltpu.SemaphoreType.DMA,
        pltpu.SemaphoreType.DMA,
    ],
    compiler_params=pltpu.CompilerParams(use_tc_tiling_on_sc=False,
                                        needs_layout_passes=False),
)(_sc_body)


def _proj_body(xr_ref, xp_ref, wr_ref, br_ref, wp_ref, bp_ref,
               hr_ref, hp_ref):
    hr = jnp.dot(xr_ref[...], wr_ref[...],
                 preferred_element_type=jnp.float32) + br_ref[...]
    hp = jnp.dot(xp_ref[...], wp_ref[...],
                 preferred_element_type=jnp.float32) + bp_ref[...]
    hr_ref[0] = hr[:, :HH]
    hr_ref[1] = hr[:, HH:]
    hp_ref[0] = hp[:, :HH]
    hp_ref[1] = hp[:, HH:]


def _proj(x_review, x_product, W_review, b_review, W_product, b_product):
    blk = lambda: pl.BlockSpec((BB, DIN), lambda i: (i, 0))
    full = lambda shape: pl.BlockSpec(shape, lambda i: tuple(0 for _ in shape))
    outb = lambda: pl.BlockSpec((2, BB, HH), lambda i: (0, i, 0))
    return pl.pallas_call(
        _proj_body,
        grid=(NB,),
        in_specs=[blk(), blk(), full((DIN, H)), full((1, H)),
                  full((DIN, H)), full((1, H))],
        out_specs=[outb(), outb()],
        out_shape=[jax.ShapeDtypeStruct((2, NR, HH), jnp.float32)] * 2,
    )(x_review, x_product, W_review, b_review.reshape(1, H),
      W_product, b_product.reshape(1, H))


def _combine_body(split_out, relu,
                  hr_ref, hp_ref, ap_ref, ar_ref, cnt_ref,
                  wroot_ref, wrel_ref, b_ref,
                  *out_refs):
    inv_p = 1.0 / jnp.maximum(cnt_ref[0][:, 0:1], 1.0)
    inv_r = 1.0 / jnp.maximum(cnt_ref[1][:, 0:1], 1.0)
    h_rev = jnp.concatenate([hr_ref[0], hr_ref[1]], axis=1)
    h_prod = jnp.concatenate([hp_ref[0], hp_ref[1]], axis=1)
    agg_p = jnp.concatenate([ap_ref[0], ap_ref[1]], axis=1) * inv_p
    agg_r = jnp.concatenate([ar_ref[0], ar_ref[1]], axis=1) * inv_r
    wroot = wroot_ref[...]
    b = b_ref[...]
    out_rev = (jnp.dot(h_rev, wroot, preferred_element_type=jnp.float32) + b
               + jnp.dot(agg_r, wrel_ref[1], preferred_element_type=jnp.float32))
    out_prod = (jnp.dot(h_prod, wroot, preferred_element_type=jnp.float32) + b
                + jnp.dot(agg_p, wrel_ref[0], preferred_element_type=jnp.float32))
    if relu:
        out_rev = jnp.maximum(out_rev, 0.0)
        out_prod = jnp.maximum(out_prod, 0.0)
    if split_out:
        out_refs[0][0] = out_rev[:, :HH]
        out_refs[0][1] = out_rev[:, HH:]
        out_refs[1][0] = out_prod[:, :HH]
        out_refs[1][1] = out_prod[:, HH:]
    else:
        out_refs[0][0] = out_rev
        out_refs[0][1] = out_prod


def _combine(hr3, hp3, aggp3, aggr3, cnt3, w_rel, w_root, b,
             split_out, relu):
    dout = w_root.shape[1]
    pblk = lambda w: pl.BlockSpec((2, BB, w), lambda i: (0, i, 0))
    full = lambda shape: pl.BlockSpec(shape, lambda i: tuple(0 for _ in shape))
    if split_out:
        out_specs = [pblk(HH), pblk(HH)]
        out_shape = [jax.ShapeDtypeStruct((2, NR, HH), jnp.float32)] * 2
    else:
        out_specs = [pl.BlockSpec((2, BB, dout), lambda i: (0, i, 0))]
        out_shape = [jax.ShapeDtypeStruct((2, NR, dout), jnp.float32)]
    return pl.pallas_call(
        functools.partial(_combine_body, split_out, relu),
        grid=(NB,),
        in_specs=[pblk(HH)] * 5
                 + [full((H, dout)), full((2, H, dout)), full((1, dout))],
        out_specs=out_specs,
        out_shape=out_shape,
    )(hr3, hp3, aggp3, aggr3, cnt3, w_root, w_rel, b.reshape(1, dout))


def kernel(x_review, x_product, edge_index_r2p, edge_index_p2r,
           W_review, b_review, W_product, b_product,
           conv1_w_rel, conv1_w_root, conv1_b,
           conv2_w_rel, conv2_w_root, conv2_b):
    src_r2 = edge_index_r2p[0].reshape(E // EC, EC)
    dst_p2 = (edge_index_r2p[1] - NR).reshape(E // EC, EC)
    src_p2 = (edge_index_p2r[0] - NR).reshape(E // EC, EC)
    dst_r2 = edge_index_p2r[1].reshape(E // EC, EC)
    ones = jnp.ones((EC, HH), jnp.float32)
    zrows = jnp.zeros((DR, HH), jnp.float32)
    flag1 = jnp.ones((16,), jnp.int32)
    flag0 = jnp.zeros((16,), jnp.int32)

    hr3, hp3 = _proj(x_review, x_product, W_review, b_review,
                     W_product, b_product)
    cnt3, aggp3, aggr3 = _sc_call(hr3, hp3, src_r2, dst_p2, src_p2, dst_r2,
                                  ones, zrows, flag1)
    hr3b, hp3b = _combine(hr3, hp3, aggp3, aggr3, cnt3,
                          conv1_w_rel, conv1_w_root, conv1_b,
                          split_out=True, relu=True)
    _, aggp3b, aggr3b = _sc_call(hr3b, hp3b, src_r2, dst_p2, src_p2, dst_r2,
                                 ones, zrows, flag0)
    (out,) = _combine(hr3b, hp3b, aggp3b, aggr3b, cnt3,
                      conv2_w_rel, conv2_w_root, conv2_b,
                      split_out=False, relu=False)
    return out.reshape(2 * NR, OUT)
